# baseline (speedup 1.0000x reference)
import functools

import jax
import jax.numpy as jnp
from jax import lax
from jax.experimental import pallas as pl
from jax.experimental.pallas import tpu as pltpu
from jax.experimental.pallas import tpu_sc as plsc

D_MODEL = 64
SCALE = 8.0
NUM_CORES = 2
NUM_SUBCORES = 16
NUM_WORKERS = NUM_CORES * NUM_SUBCORES
LANES = 128
NBUF = 2
PACK_ROWS = 1600
HALF = PACK_ROWS // 2


def _pack_body(t_ref, o_ref):
    o_ref[:, :D_MODEL] = t_ref[:HALF, :] * SCALE
    o_ref[:, D_MODEL:] = t_ref[HALF:, :] * SCALE


@functools.lru_cache(maxsize=None)
def _pack_build(vocab: int):
    return pl.pallas_call(
        _pack_body,
        grid=(vocab // PACK_ROWS,),
        in_specs=[pl.BlockSpec((PACK_ROWS, D_MODEL), lambda i: (i, 0))],
        out_specs=pl.BlockSpec((HALF, 2 * D_MODEL), lambda i: (i, 0)),
        out_shape=jax.ShapeDtypeStruct((vocab // 2, 2 * D_MODEL),
                                       jnp.float32),
    )


@functools.lru_cache(maxsize=None)
def _build(n_t: int, n_b: int, vocab: int):
    n_bblk = n_b // LANES
    assert n_bblk == NUM_WORKERS
    mesh = plsc.VectorSubcoreMesh(core_axis_name="c", subcore_axis_name="s")

    @functools.partial(
        pl.kernel,
        mesh=mesh,
        out_type=jax.ShapeDtypeStruct(
            (n_t, D_MODEL // 8, n_bblk, 8, LANES), jnp.float32
        ),
        scratch_types=[
            pltpu.VMEM((n_t, LANES), jnp.int32),
            pltpu.VMEM((n_t, LANES), jnp.int32),
            pltpu.VMEM((NBUF, LANES, 2 * D_MODEL), jnp.float32),
            pltpu.VMEM((NBUF, D_MODEL // 8, 8, LANES), jnp.float32),
            pltpu.SemaphoreType.DMA((NBUF,)),
            pltpu.SemaphoreType.DMA((NBUF,)),
        ],
        compiler_params=pltpu.CompilerParams(
            use_tc_tiling_on_sc=False, needs_layout_passes=False
        ),
    )
    def emb_kernel(xj_hbm, xp_hbm, pairs_hbm, out_hbm, idx_v, par_v, rows_v,
                   trans_v, in_sem, out_sem):
        wid = lax.axis_index("s") * NUM_CORES + lax.axis_index("c")
        pltpu.sync_copy(xj_hbm.at[wid], idx_v)
        pltpu.sync_copy(xp_hbm.at[wid], par_v)

        def gather(t, slot):
            pltpu.async_copy(pairs_hbm.at[idx_v.at[t]], rows_v.at[slot],
                             in_sem.at[slot])

        def gather_wait(t, slot):
            pltpu.make_async_copy(pairs_hbm.at[idx_v.at[t]], rows_v.at[slot],
                                  in_sem.at[slot]).wait()

        def scatter(t, slot):
            pltpu.async_copy(trans_v.at[slot], out_hbm.at[t, :, wid],
                             out_sem.at[slot])

        def scatter_wait(t, slot):
            pltpu.make_async_copy(trans_v.at[slot], out_hbm.at[t, :, wid],
                                  out_sem.at[slot]).wait()

        for b in range(NBUF):
            gather(b, b)

        jvec = lax.iota(jnp.int32, 16)

        def pair_body(p, carry):
            for b in range(NBUF):
                t = p * NBUF + b
                gather_wait(t, b)

                @pl.when(p > 0)
                def _():
                    scatter_wait(t, b)

                par8 = [par_v[t, pl.ds(l0, 16)] for l0 in range(0, LANES, 16)]

                def diag_body(s, c):
                    perm = jnp.bitwise_and(jvec + s, 15)
                    vals = []
                    for d0 in range(0, D_MODEL, 16):
                        dcol = d0 + perm
                        dhi = jnp.right_shift(dcol, 3)
                        dlo = jnp.bitwise_and(dcol, 7)
                        for li, l0 in enumerate(range(0, LANES, 16)):
                            lrow = l0 + jvec
                            vals.append(
                                (plsc.load_gather(rows_v.at[b],
                                                  [lrow, dcol + par8[li]]),
                                 dhi, dlo, lrow))
                    for val, dhi, dlo, lrow in vals:
                        plsc.store_scatter(trans_v.at[b], [dhi, dlo, lrow],
                                           val)
                    return c

                lax.fori_loop(0, 16, diag_body, 0)

                @pl.when(t + NBUF < n_t)
                def _():
                    gather(t + NBUF, b)

                scatter(t, b)
            return carry

        lax.fori_loop(0, n_t // NBUF, pair_body, 0)

        for b in range(NBUF):
            scatter_wait(n_t - NBUF + b, b)

    return emb_kernel


def kernel(x, table):
    n_b, n_t = x.shape
    vocab = table.shape[0]
    xw = jnp.transpose(x, (1, 0)).reshape(n_t, NUM_WORKERS, LANES)
    xw = jnp.transpose(xw, (1, 0, 2))
    xj = (xw // PACK_ROWS) * HALF + jnp.remainder(xw, HALF)
    xp = jnp.left_shift(jnp.remainder(xw, PACK_ROWS) // HALF, 6)
    pairs = _pack_build(vocab)(table)
    out5 = _build(n_t, n_b, vocab)(xj, xp, pairs)
    return out5.transpose(2, 4, 0, 1, 3).reshape(n_b, n_t, D_MODEL)


# speedup vs baseline: 1.4849x; 1.4849x over previous
import functools

import jax
import jax.numpy as jnp
from jax import lax
from jax.experimental import pallas as pl
from jax.experimental.pallas import tpu as pltpu
from jax.experimental.pallas import tpu_sc as plsc

D_MODEL = 64
SCALE = 8.0
NUM_CORES = 2
NUM_SUBCORES = 16
NUM_WORKERS = NUM_CORES * NUM_SUBCORES
LANES = 128
NBUF = 4


@functools.lru_cache(maxsize=None)
def _build(n_t: int, n_b: int, vocab: int):
    n_bblk = n_b // LANES
    assert n_bblk == NUM_WORKERS
    mesh = plsc.VectorSubcoreMesh(core_axis_name="c", subcore_axis_name="s")

    @functools.partial(
        pl.kernel,
        mesh=mesh,
        out_type=jax.ShapeDtypeStruct(
            (n_t, D_MODEL // 8, n_bblk, 8, LANES), jnp.float32
        ),
        scratch_types=[
            pltpu.VMEM((n_t, LANES), jnp.int32),
            pltpu.VMEM((NBUF, LANES, D_MODEL), jnp.float32),
            pltpu.VMEM((NBUF, D_MODEL // 8, 8, LANES), jnp.float32),
            pltpu.SemaphoreType.DMA((NBUF,)),
            pltpu.SemaphoreType.DMA((NBUF,)),
        ],
        compiler_params=pltpu.CompilerParams(
            use_tc_tiling_on_sc=False, needs_layout_passes=False
        ),
    )
    def emb_kernel(xw_hbm, tbl_hbm, out_hbm, idx_v, rows_v, trans_v,
                   in_sem, out_sem):
        wid = lax.axis_index("s") * NUM_CORES + lax.axis_index("c")
        pltpu.sync_copy(xw_hbm.at[wid], idx_v)

        def gather(t, slot):
            pltpu.async_copy(tbl_hbm.at[idx_v.at[t]], rows_v.at[slot],
                             in_sem.at[slot])

        def gather_wait(t, slot):
            pltpu.make_async_copy(tbl_hbm.at[idx_v.at[t]], rows_v.at[slot],
                                  in_sem.at[slot]).wait()

        def scatter(t, slot):
            pltpu.async_copy(trans_v.at[slot], out_hbm.at[t, :, wid],
                             out_sem.at[slot])

        def scatter_wait(t, slot):
            pltpu.make_async_copy(trans_v.at[slot], out_hbm.at[t, :, wid],
                                  out_sem.at[slot]).wait()

        for b in range(NBUF):
            gather(b, b)

        jvec = lax.iota(jnp.int32, 16)

        def pair_body(p, carry):
            for b in range(NBUF):
                t = p * NBUF + b
                gather_wait(t, b)

                @pl.when(p > 0)
                def _():
                    scatter_wait(t, b)

                def diag_body(s, c):
                    perm = jnp.bitwise_and(jvec + s, 15)
                    vals = []
                    for d0 in range(0, D_MODEL, 16):
                        dcol = d0 + perm
                        dhi = jnp.right_shift(dcol, 3)
                        dlo = jnp.bitwise_and(dcol, 7)
                        for l0 in range(0, LANES, 16):
                            lrow = l0 + jvec
                            vals.append(
                                (plsc.load_gather(rows_v.at[b], [lrow, dcol]),
                                 dhi, dlo, lrow))
                    for val, dhi, dlo, lrow in vals:
                        plsc.store_scatter(trans_v.at[b], [dhi, dlo, lrow],
                                           val * SCALE)
                    return c

                lax.fori_loop(0, 16, diag_body, 0, unroll=2)

                @pl.when(t + NBUF < n_t)
                def _():
                    gather(t + NBUF, b)

                scatter(t, b)
            return carry

        lax.fori_loop(0, n_t // NBUF, pair_body, 0)

        for b in range(NBUF):
            scatter_wait(n_t - NBUF + b, b)

    return emb_kernel


def kernel(x, table):
    n_b, n_t = x.shape
    vocab = table.shape[0]
    xw = jnp.transpose(x, (1, 0)).reshape(n_t, NUM_WORKERS, LANES)
    xw = jnp.transpose(xw, (1, 0, 2))
    out5 = _build(n_t, n_b, vocab)(xw, table)
    return out5.transpose(2, 4, 0, 1, 3).reshape(n_b, n_t, D_MODEL)
